# baseline (speedup 1.0000x reference)
import functools

import jax
import jax.numpy as jnp
from jax import lax
from jax.experimental import pallas as pl
from jax.experimental.pallas import tpu as pltpu
from jax.experimental.pallas import tpu_sc as plsc

N_NODES = 50000
N_EDGES = 1600000
NUM_GRAPHS = 128
FEATS_DIM = 5
POS_DIM = 3
M_DIM = 16

BE = 2000
BN = 2000
GE = N_EDGES // BE
GN = N_NODES // BN


def _silu(v):
    return v * jax.nn.sigmoid(v)


def _edge_body(xs_ref, xd_ref, ea_ref, w1t_ref, b1_ref, w2t_ref, b2_ref,
               swt_ref, sb_ref, cw1t_ref, cb1_ref, cw2t_ref, cb2_ref,
               cs_ref, msg_ref):
    xs = xs_ref[...]
    xd = xd_ref[...]
    ea = ea_ref[...]
    rel = xs[:, 0:POS_DIM] - xd[:, 0:POS_DIM]
    rd = jnp.sum(rel * rel, axis=1, keepdims=True)
    m_in = jnp.concatenate([xd[:, POS_DIM:], xs[:, POS_DIM:], ea, rd], axis=1)
    h = _silu(jnp.dot(m_in, w1t_ref[...], preferred_element_type=jnp.float32, precision=lax.Precision.HIGHEST)
              + b1_ref[...])
    mij = _silu(jnp.dot(h, w2t_ref[...], preferred_element_type=jnp.float32, precision=lax.Precision.HIGHEST)
                + b2_ref[...])
    ch = _silu(jnp.dot(mij, cw1t_ref[...], preferred_element_type=jnp.float32, precision=lax.Precision.HIGHEST)
               + cb1_ref[...])
    cwij = (jnp.dot(ch, cw2t_ref[...], preferred_element_type=jnp.float32, precision=lax.Precision.HIGHEST)
            + cb2_ref[...])
    nrm = jnp.sqrt(jnp.maximum(rd, 1e-16))
    reln = rel / jnp.maximum(nrm, 1e-8) * cs_ref[0, 0]
    wv = cwij * reln
    gate = jax.nn.sigmoid(
        jnp.dot(mij, swt_ref[...], preferred_element_type=jnp.float32, precision=lax.Precision.HIGHEST)
        + sb_ref[...])
    msg_ref[...] = jnp.concatenate(
        [mij * gate, wv, jnp.zeros((wv.shape[0], 5), jnp.float32)], axis=1)


def _edge_call(xs, xd, ea, p):
    full = lambda shp: pl.BlockSpec(shp, lambda i: (0, 0))
    return pl.pallas_call(
        _edge_body,
        grid=(GE,),
        in_specs=[
            pl.BlockSpec((BE, 8), lambda i: (i, 0)),
            pl.BlockSpec((BE, 8), lambda i: (i, 0)),
            pl.BlockSpec((BE, 4), lambda i: (i, 0)),
            full((15, 30)), full((1, 30)),
            full((30, 16)), full((1, 16)),
            full((16, 1)), full((1, 1)),
            full((16, 64)), full((1, 64)),
            full((64, 1)), full((1, 1)),
            full((1, 1)),
        ],
        out_specs=pl.BlockSpec((BE, 24), lambda i: (i, 0)),
        out_shape=jax.ShapeDtypeStruct((N_EDGES, 24), jnp.float32),
    )(xs, xd, ea,
      p["edge_w1"].T, p["edge_b1"][None, :],
      p["edge_w2"].T, p["edge_b2"][None, :],
      p["soft_w"].T, p["soft_b"][None, :],
      p["coors_w1"].T, p["coors_b1"][None, :],
      p["coors_w2"].T, p["coors_b2"][None, :],
      p["coors_scale"].reshape(1, 1))


SC_ROWS = N_EDGES // 128
SC_K = 8
SC_FULL = SC_ROWS // SC_K
SC_T = SC_FULL // 32
SC_REM = SC_FULL - SC_T * 32
SC_TAIL = SC_ROWS - SC_FULL * SC_K
N_PAD = 50048
SC_NSLICE = N_PAD // 16


def _sc_scatter(msgc, dst2d, z24):
    mesh = plsc.VectorSubcoreMesh(core_axis_name="c", subcore_axis_name="s")

    @functools.partial(
        pl.kernel,
        out_type=jax.ShapeDtypeStruct((2, N_PAD, 24), jnp.float32),
        mesh=mesh,
        scratch_types=[
            pltpu.VMEM((SC_K, 128), jnp.int32),
            pltpu.VMEM((SC_K * 128, 24), jnp.float32),
            pltpu.VMEM_SHARED((N_PAD, 24), jnp.float32),
        ],
        compiler_params=pltpu.CompilerParams(use_tc_tiling_on_sc=False),
    )
    def k(msg_hbm, dst_hbm, z_hbm, out_hbm, idx_v, data_v, acc_sh):
        c = lax.axis_index("c")
        s = lax.axis_index("s")
        w = c * 16 + s
        r0 = s * SC_NSLICE
        pltpu.sync_copy(z_hbm.at[pl.ds(r0, SC_NSLICE)],
                        acc_sh.at[pl.ds(r0, SC_NSLICE)])
        plsc.subcore_barrier()

        def chunk(q, nrows):
            row = q * SC_K
            e0 = row * 128
            pltpu.sync_copy(dst_hbm.at[pl.ds(row, nrows)],
                            idx_v.at[pl.ds(0, nrows)])
            pltpu.sync_copy(msg_hbm.at[pl.ds(e0, nrows * 128)],
                            data_v.at[pl.ds(0, nrows * 128)])
            for j in range(nrows):
                pltpu.sync_copy(data_v.at[pl.ds(j * 128, 128)],
                                acc_sh.at[idx_v.at[j]], add=True)

        def body(t, carry):
            chunk(w + 32 * t, SC_K)
            return carry

        lax.fori_loop(0, SC_T, body, 0)

        @pl.when(w < SC_REM)
        def _():
            chunk(SC_T * 32 + w, SC_K)

        @pl.when(w == 31)
        def _():
            chunk(SC_FULL, SC_TAIL)

        plsc.subcore_barrier()
        pltpu.sync_copy(acc_sh.at[pl.ds(r0, SC_NSLICE)],
                        out_hbm.at[c, pl.ds(r0, SC_NSLICE)])

    return k(msgc, dst2d, z24)


def _onehot(b):
    g = lax.broadcasted_iota(jnp.int32, (b.shape[0], NUM_GRAPHS), 1)
    return (b == g).astype(jnp.float32)


def _segdot(oh, vals):
    return lax.dot_general(oh, vals, (((0,), (0,)), ((), ())),
                           preferred_element_type=jnp.float32, precision=lax.Precision.HIGHEST)


def _ns_body(x_ref, b_ref, out_ref):
    @pl.when(pl.program_id(0) == 0)
    def _():
        out_ref[...] = jnp.zeros_like(out_ref)

    feats = x_ref[...][:, POS_DIM:]
    v1 = jnp.sum(feats, axis=1, keepdims=True)
    v2 = jnp.sum(feats * feats, axis=1, keepdims=True)
    ones = jnp.ones_like(v1)
    vals = jnp.concatenate(
        [v1, v2, ones, jnp.zeros((v1.shape[0], 5), jnp.float32)], axis=1)
    out_ref[...] += _segdot(_onehot(b_ref[...]), vals)


def _ns_call(x, b2):
    return pl.pallas_call(
        _ns_body,
        grid=(GN,),
        in_specs=[
            pl.BlockSpec((BN, 8), lambda i: (i, 0)),
            pl.BlockSpec((BN, 1), lambda i: (i, 0)),
        ],
        out_specs=pl.BlockSpec((NUM_GRAPHS, 8), lambda i: (0, 0)),
        out_shape=jax.ShapeDtypeStruct((NUM_GRAPHS, 8), jnp.float32),
    )(x, b2)


def _nu_body(x_ref, acc_ref, b_ref, ls_ref, lnw_ref, lnb_ref,
             nw1t_ref, nb1_ref, nw2t_ref, nb2_ref, xp_ref, gs_ref):
    x = x_ref[...]
    acc = jnp.sum(acc_ref[...], axis=0)
    m_i = acc[:, 0:16]
    coors = x[:, 0:POS_DIM] + acc[:, 16:16 + POS_DIM]
    feats = x[:, POS_DIM:]

    ls = ls_ref[...]
    normv = jnp.maximum(ls[:, 2:3], 1.0) * float(FEATS_DIM)
    m = ls[:, 0:1] / normv
    var = ls[:, 1:2] / normv - m * m
    inv = lax.rsqrt(var + 1e-5)
    pg = jnp.concatenate(
        [m, inv, jnp.zeros((NUM_GRAPHS, 6), jnp.float32)], axis=1)
    pn = jnp.dot(_onehot(b_ref[...]), pg, preferred_element_type=jnp.float32, precision=lax.Precision.HIGHEST)
    feats_n = (feats - pn[:, 0:1]) * pn[:, 1:2] * lnw_ref[...] + lnb_ref[...]

    h2in = jnp.concatenate([feats_n, m_i], axis=1)
    h2 = _silu(jnp.dot(h2in, nw1t_ref[...], preferred_element_type=jnp.float32, precision=lax.Precision.HIGHEST)
               + nb1_ref[...])
    fo = feats + jnp.dot(h2, nw2t_ref[...],
                         preferred_element_type=jnp.float32, precision=lax.Precision.HIGHEST) + nb2_ref[...]
    xp = jnp.concatenate([coors, fo], axis=1)
    xp_ref[...] = xp

    @pl.when(pl.program_id(0) == 0)
    def _():
        gs_ref[...] = jnp.zeros_like(gs_ref)

    s1 = jnp.sum(xp, axis=0, keepdims=True)
    s2 = jnp.sum(xp * xp, axis=0, keepdims=True)
    gs_ref[...] += jnp.concatenate(
        [s1, s2, jnp.zeros((6, 8), jnp.float32)], axis=0)


def _nu_call(x, acc, b2, lnstats, p):
    full = lambda shp: pl.BlockSpec(shp, lambda i: tuple(0 for _ in shp))
    a = acc.shape[0]
    return pl.pallas_call(
        _nu_body,
        grid=(GN,),
        in_specs=[
            pl.BlockSpec((BN, 8), lambda i: (i, 0)),
            pl.BlockSpec((a, BN, 24), lambda i: (0, i, 0)),
            pl.BlockSpec((BN, 1), lambda i: (i, 0)),
            full((NUM_GRAPHS, 8)),
            full((1, 5)), full((1, 5)),
            full((21, 10)), full((1, 10)),
            full((10, 5)), full((1, 5)),
        ],
        out_specs=[
            pl.BlockSpec((BN, 8), lambda i: (i, 0)),
            pl.BlockSpec((8, 8), lambda i: (0, 0)),
        ],
        out_shape=[
            jax.ShapeDtypeStruct((N_NODES, 8), jnp.float32),
            jax.ShapeDtypeStruct((8, 8), jnp.float32),
        ],
    )(x, acc, b2, lnstats,
      p["ln_w"][None, :], p["ln_b"][None, :],
      p["node_w1"].T, p["node_b1"][None, :],
      p["node_w2"].T, p["node_b2"][None, :])


def _ga_body(xp_ref, gs_ref, b_ref, gw_ref, gb_ref, gms_ref,
             y_ref, st_ref, *, relu, last):
    gs = gs_ref[...]
    nf = float(N_NODES)
    mean = gs[0:1, :] / nf
    e2 = gs[1:2, :] / nf
    c = mean * gms_ref[...]
    var = e2 - 2.0 * mean * c + c * c
    y = gw_ref[...] * (xp_ref[...] - c) / jnp.sqrt(var + 1e-5) + gb_ref[...]
    if relu:
        y = jnp.maximum(y, 0.0)
    y_ref[...] = y

    @pl.when(pl.program_id(0) == 0)
    def _():
        st_ref[...] = jnp.zeros_like(st_ref)

    oh = _onehot(b_ref[...])
    if last:
        st_ref[...] += _segdot(oh, y)
    else:
        feats = y[:, POS_DIM:]
        v1 = jnp.sum(feats, axis=1, keepdims=True)
        v2 = jnp.sum(feats * feats, axis=1, keepdims=True)
        ones = jnp.ones_like(v1)
        vals = jnp.concatenate(
            [v1, v2, ones, jnp.zeros((v1.shape[0], 5), jnp.float32)], axis=1)
        st_ref[...] += _segdot(oh, vals)


def _ga_call(xp, gstats, b2, g, relu, last):
    full = lambda shp: pl.BlockSpec(shp, lambda i: (0, 0))
    return pl.pallas_call(
        functools.partial(_ga_body, relu=relu, last=last),
        grid=(GN,),
        in_specs=[
            pl.BlockSpec((BN, 8), lambda i: (i, 0)),
            full((8, 8)),
            pl.BlockSpec((BN, 1), lambda i: (i, 0)),
            full((1, 8)), full((1, 8)), full((1, 8)),
        ],
        out_specs=[
            pl.BlockSpec((BN, 8), lambda i: (i, 0)),
            pl.BlockSpec((NUM_GRAPHS, 8), lambda i: (0, 0)),
        ],
        out_shape=[
            jax.ShapeDtypeStruct((N_NODES, 8), jnp.float32),
            jax.ShapeDtypeStruct((NUM_GRAPHS, 8), jnp.float32),
        ],
    )(xp, gstats, b2,
      g["weight"][None, :], g["bias"][None, :], g["mean_scale"][None, :])


def _head_body(pool_ref, cnt_ref, w1t_ref, b1_ref, w2t_ref, b2_ref, out_ref):
    h = pool_ref[...] / jnp.maximum(cnt_ref[...], 1.0)
    h1 = jnp.maximum(
        jnp.dot(h, w1t_ref[...], preferred_element_type=jnp.float32, precision=lax.Precision.HIGHEST)
        + b1_ref[...], 0.0)
    out_ref[...] = (jnp.dot(h1, w2t_ref[...],
                            preferred_element_type=jnp.float32, precision=lax.Precision.HIGHEST) + b2_ref[...])


def _head_call(pool, cnt, fc):
    (w1, b1), (w2, b2) = fc
    full = lambda shp: pl.BlockSpec(shp, lambda: (0, 0))
    return pl.pallas_call(
        _head_body,
        in_specs=[full((NUM_GRAPHS, 8)), full((NUM_GRAPHS, 1)),
                  full((8, 32)), full((1, 32)),
                  full((32, 10)), full((1, 10))],
        out_specs=full((NUM_GRAPHS, 10)),
        out_shape=jax.ShapeDtypeStruct((NUM_GRAPHS, 10), jnp.float32),
    )(pool, cnt, w1.T, b1[None, :], w2.T, b2[None, :])


def kernel(x, edge_index, batch, edge_attr, params):
    src = edge_index[0]
    dst = edge_index[1]
    b2 = batch[:, None]
    dst2d = dst.reshape(SC_ROWS, 128)
    z24 = jnp.zeros((N_PAD, 24), jnp.float32)

    lnstats = _ns_call(x, b2)
    cnt = lnstats[:, 2:3]

    x_cur = x
    for i in range(3):
        p = params["layers"][i]
        xs = jnp.take(x_cur, src, axis=0)
        xd = jnp.take(x_cur, dst, axis=0)
        msgc = _edge_call(xs, xd, edge_attr, p)
        acc = _sc_scatter(msgc, dst2d, z24)
        xp, gstats = _nu_call(x_cur, acc, b2, lnstats, p)
        last = i == 2
        x_cur, aux = _ga_call(xp, gstats, b2, params["gn"][i],
                              relu=not last, last=last)
        if not last:
            lnstats = aux
    return _head_call(aux, cnt, params["fc"])


# speedup vs baseline: 1.6610x; 1.6610x over previous
import functools

import jax
import jax.numpy as jnp
from jax import lax
from jax.experimental import pallas as pl
from jax.experimental.pallas import tpu as pltpu
from jax.experimental.pallas import tpu_sc as plsc

N_NODES = 50000
N_EDGES = 1600000
NUM_GRAPHS = 128
FEATS_DIM = 5
POS_DIM = 3
M_DIM = 16

BE = 2000
BN = 2000
GE = N_EDGES // BE
GN = N_NODES // BN


def _silu(v):
    return v * jax.nn.sigmoid(v)


def _edge_body(xs_ref, xd_ref, ea_ref, w1t_ref, b1_ref, w2t_ref, b2_ref,
               swt_ref, sb_ref, cw1t_ref, cb1_ref, cw2t_ref, cb2_ref,
               cs_ref, msg_ref):
    xs = xs_ref[...]
    xd = xd_ref[...]
    ea = ea_ref[...]
    rel = xs[:, 0:POS_DIM] - xd[:, 0:POS_DIM]
    rd = jnp.sum(rel * rel, axis=1, keepdims=True)
    m_in = jnp.concatenate([xd[:, POS_DIM:], xs[:, POS_DIM:], ea, rd], axis=1)
    h = _silu(jnp.dot(m_in, w1t_ref[...], preferred_element_type=jnp.float32, precision=lax.Precision.HIGHEST)
              + b1_ref[...])
    mij = _silu(jnp.dot(h, w2t_ref[...], preferred_element_type=jnp.float32, precision=lax.Precision.HIGHEST)
                + b2_ref[...])
    ch = _silu(jnp.dot(mij, cw1t_ref[...], preferred_element_type=jnp.float32, precision=lax.Precision.HIGHEST)
               + cb1_ref[...])
    cwij = (jnp.dot(ch, cw2t_ref[...], preferred_element_type=jnp.float32, precision=lax.Precision.HIGHEST)
            + cb2_ref[...])
    nrm = jnp.sqrt(jnp.maximum(rd, 1e-16))
    reln = rel / jnp.maximum(nrm, 1e-8) * cs_ref[0, 0]
    wv = cwij * reln
    gate = jax.nn.sigmoid(
        jnp.dot(mij, swt_ref[...], preferred_element_type=jnp.float32, precision=lax.Precision.HIGHEST)
        + sb_ref[...])
    msg_ref[...] = jnp.concatenate(
        [mij * gate, wv, jnp.zeros((wv.shape[0], 5), jnp.float32)], axis=1)


def _edge_call(xs, xd, ea, p):
    full = lambda shp: pl.BlockSpec(shp, lambda i: (0, 0))
    return pl.pallas_call(
        _edge_body,
        grid=(GE,),
        in_specs=[
            pl.BlockSpec((BE, 8), lambda i: (i, 0)),
            pl.BlockSpec((BE, 8), lambda i: (i, 0)),
            pl.BlockSpec((BE, 4), lambda i: (i, 0)),
            full((15, 30)), full((1, 30)),
            full((30, 16)), full((1, 16)),
            full((16, 1)), full((1, 1)),
            full((16, 64)), full((1, 64)),
            full((64, 1)), full((1, 1)),
            full((1, 1)),
        ],
        out_specs=pl.BlockSpec((BE, 24), lambda i: (i, 0)),
        out_shape=jax.ShapeDtypeStruct((N_EDGES, 24), jnp.float32),
    )(xs, xd, ea,
      p["edge_w1"].T, p["edge_b1"][None, :],
      p["edge_w2"].T, p["edge_b2"][None, :],
      p["soft_w"].T, p["soft_b"][None, :],
      p["coors_w1"].T, p["coors_b1"][None, :],
      p["coors_w2"].T, p["coors_b2"][None, :],
      p["coors_scale"].reshape(1, 1))


SC_ROWS = N_EDGES // 128
SC_K = 8
SC_FULL = SC_ROWS // SC_K
SC_T = SC_FULL // 32
SC_REM = SC_FULL - SC_T * 32
SC_TAIL = SC_ROWS - SC_FULL * SC_K
N_PAD = 50048
SC_NSLICE = N_PAD // 16


def _sc_scatter(msgc, dst2d, z24):
    mesh = plsc.VectorSubcoreMesh(core_axis_name="c", subcore_axis_name="s")

    @functools.partial(
        pl.kernel,
        out_type=jax.ShapeDtypeStruct((2, N_PAD, 24), jnp.float32),
        mesh=mesh,
        scratch_types=[
            pltpu.VMEM((SC_K, 128), jnp.int32),
            pltpu.VMEM((SC_K * 128, 24), jnp.float32),
            pltpu.VMEM_SHARED((N_PAD, 24), jnp.float32),
        ],
        compiler_params=pltpu.CompilerParams(use_tc_tiling_on_sc=False),
    )
    def k(msg_hbm, dst_hbm, z_hbm, out_hbm, idx_v, data_v, acc_sh):
        c = lax.axis_index("c")
        s = lax.axis_index("s")
        w = c * 16 + s
        r0 = s * SC_NSLICE
        pltpu.sync_copy(z_hbm.at[pl.ds(r0, SC_NSLICE)],
                        acc_sh.at[pl.ds(r0, SC_NSLICE)])
        plsc.subcore_barrier()

        def chunk(q, nrows):
            row = q * SC_K
            e0 = row * 128
            pltpu.sync_copy(dst_hbm.at[pl.ds(row, nrows)],
                            idx_v.at[pl.ds(0, nrows)])
            pltpu.sync_copy(msg_hbm.at[pl.ds(e0, nrows * 128)],
                            data_v.at[pl.ds(0, nrows * 128)])
            for j in range(nrows):
                pltpu.sync_copy(data_v.at[pl.ds(j * 128, 128)],
                                acc_sh.at[idx_v.at[j]], add=True)

        def body(t, carry):
            chunk(w + 32 * t, SC_K)
            return carry

        lax.fori_loop(0, SC_T, body, 0)

        @pl.when(w < SC_REM)
        def _():
            chunk(SC_T * 32 + w, SC_K)

        @pl.when(w == 31)
        def _():
            chunk(SC_FULL, SC_TAIL)

        plsc.subcore_barrier()
        pltpu.sync_copy(acc_sh.at[pl.ds(r0, SC_NSLICE)],
                        out_hbm.at[c, pl.ds(r0, SC_NSLICE)])

    return k(msgc, dst2d, z24)


def _sc_gather(x, src2d, dst2d):
    mesh = plsc.VectorSubcoreMesh(core_axis_name="c", subcore_axis_name="s")

    @functools.partial(
        pl.kernel,
        out_type=[jax.ShapeDtypeStruct((N_EDGES, 8), jnp.float32),
                  jax.ShapeDtypeStruct((N_EDGES, 8), jnp.float32)],
        mesh=mesh,
        scratch_types=[
            pltpu.VMEM((SC_K, 128), jnp.int32),
            pltpu.VMEM((SC_K, 128), jnp.int32),
            pltpu.VMEM((SC_K * 128, 8), jnp.float32),
            pltpu.VMEM((SC_K * 128, 8), jnp.float32),
            pltpu.SemaphoreType.DMA,
        ],
        compiler_params=pltpu.CompilerParams(use_tc_tiling_on_sc=False),
    )
    def k(x_hbm, src_hbm, dst_hbm, gs_hbm, gd_hbm,
          si_v, di_v, sb_v, db_v, sem):
        c = lax.axis_index("c")
        s = lax.axis_index("s")
        w = c * 16 + s

        def chunk(q, nrows):
            row = q * SC_K
            e0 = row * 128
            pltpu.sync_copy(src_hbm.at[pl.ds(row, nrows)],
                            si_v.at[pl.ds(0, nrows)])
            pltpu.sync_copy(dst_hbm.at[pl.ds(row, nrows)],
                            di_v.at[pl.ds(0, nrows)])
            hs = []
            for j in range(nrows):
                hs.append(pltpu.async_copy(
                    x_hbm.at[si_v.at[j]], sb_v.at[pl.ds(j * 128, 128)], sem))
                hs.append(pltpu.async_copy(
                    x_hbm.at[di_v.at[j]], db_v.at[pl.ds(j * 128, 128)], sem))
            for h in hs:
                h.wait()
            pltpu.sync_copy(sb_v.at[pl.ds(0, nrows * 128)],
                            gs_hbm.at[pl.ds(e0, nrows * 128)])
            pltpu.sync_copy(db_v.at[pl.ds(0, nrows * 128)],
                            gd_hbm.at[pl.ds(e0, nrows * 128)])

        def body(t, carry):
            chunk(w + 32 * t, SC_K)
            return carry

        lax.fori_loop(0, SC_T, body, 0)

        @pl.when(w < SC_REM)
        def _():
            chunk(SC_T * 32 + w, SC_K)

        @pl.when(w == 31)
        def _():
            chunk(SC_FULL, SC_TAIL)

    return k(x, src2d, dst2d)


def _onehot(b):
    g = lax.broadcasted_iota(jnp.int32, (b.shape[0], NUM_GRAPHS), 1)
    return (b == g).astype(jnp.float32)


def _segdot(oh, vals):
    return lax.dot_general(oh, vals, (((0,), (0,)), ((), ())),
                           preferred_element_type=jnp.float32, precision=lax.Precision.HIGHEST)


def _ns_body(x_ref, b_ref, out_ref):
    @pl.when(pl.program_id(0) == 0)
    def _():
        out_ref[...] = jnp.zeros_like(out_ref)

    feats = x_ref[...][:, POS_DIM:]
    v1 = jnp.sum(feats, axis=1, keepdims=True)
    v2 = jnp.sum(feats * feats, axis=1, keepdims=True)
    ones = jnp.ones_like(v1)
    vals = jnp.concatenate(
        [v1, v2, ones, jnp.zeros((v1.shape[0], 5), jnp.float32)], axis=1)
    out_ref[...] += _segdot(_onehot(b_ref[...]), vals)


def _ns_call(x, b2):
    return pl.pallas_call(
        _ns_body,
        grid=(GN,),
        in_specs=[
            pl.BlockSpec((BN, 8), lambda i: (i, 0)),
            pl.BlockSpec((BN, 1), lambda i: (i, 0)),
        ],
        out_specs=pl.BlockSpec((NUM_GRAPHS, 8), lambda i: (0, 0)),
        out_shape=jax.ShapeDtypeStruct((NUM_GRAPHS, 8), jnp.float32),
    )(x, b2)


def _nu_body(x_ref, acc_ref, b_ref, ls_ref, lnw_ref, lnb_ref,
             nw1t_ref, nb1_ref, nw2t_ref, nb2_ref, xp_ref, gs_ref):
    x = x_ref[...]
    acc = jnp.sum(acc_ref[...], axis=0)
    m_i = acc[:, 0:16]
    coors = x[:, 0:POS_DIM] + acc[:, 16:16 + POS_DIM]
    feats = x[:, POS_DIM:]

    ls = ls_ref[...]
    normv = jnp.maximum(ls[:, 2:3], 1.0) * float(FEATS_DIM)
    m = ls[:, 0:1] / normv
    var = ls[:, 1:2] / normv - m * m
    inv = lax.rsqrt(var + 1e-5)
    pg = jnp.concatenate(
        [m, inv, jnp.zeros((NUM_GRAPHS, 6), jnp.float32)], axis=1)
    pn = jnp.dot(_onehot(b_ref[...]), pg, preferred_element_type=jnp.float32, precision=lax.Precision.HIGHEST)
    feats_n = (feats - pn[:, 0:1]) * pn[:, 1:2] * lnw_ref[...] + lnb_ref[...]

    h2in = jnp.concatenate([feats_n, m_i], axis=1)
    h2 = _silu(jnp.dot(h2in, nw1t_ref[...], preferred_element_type=jnp.float32, precision=lax.Precision.HIGHEST)
               + nb1_ref[...])
    fo = feats + jnp.dot(h2, nw2t_ref[...],
                         preferred_element_type=jnp.float32, precision=lax.Precision.HIGHEST) + nb2_ref[...]
    xp = jnp.concatenate([coors, fo], axis=1)
    xp_ref[...] = xp

    @pl.when(pl.program_id(0) == 0)
    def _():
        gs_ref[...] = jnp.zeros_like(gs_ref)

    s1 = jnp.sum(xp, axis=0, keepdims=True)
    s2 = jnp.sum(xp * xp, axis=0, keepdims=True)
    gs_ref[...] += jnp.concatenate(
        [s1, s2, jnp.zeros((6, 8), jnp.float32)], axis=0)


def _nu_call(x, acc, b2, lnstats, p):
    full = lambda shp: pl.BlockSpec(shp, lambda i: tuple(0 for _ in shp))
    a = acc.shape[0]
    return pl.pallas_call(
        _nu_body,
        grid=(GN,),
        in_specs=[
            pl.BlockSpec((BN, 8), lambda i: (i, 0)),
            pl.BlockSpec((a, BN, 24), lambda i: (0, i, 0)),
            pl.BlockSpec((BN, 1), lambda i: (i, 0)),
            full((NUM_GRAPHS, 8)),
            full((1, 5)), full((1, 5)),
            full((21, 10)), full((1, 10)),
            full((10, 5)), full((1, 5)),
        ],
        out_specs=[
            pl.BlockSpec((BN, 8), lambda i: (i, 0)),
            pl.BlockSpec((8, 8), lambda i: (0, 0)),
        ],
        out_shape=[
            jax.ShapeDtypeStruct((N_NODES, 8), jnp.float32),
            jax.ShapeDtypeStruct((8, 8), jnp.float32),
        ],
    )(x, acc, b2, lnstats,
      p["ln_w"][None, :], p["ln_b"][None, :],
      p["node_w1"].T, p["node_b1"][None, :],
      p["node_w2"].T, p["node_b2"][None, :])


def _ga_body(xp_ref, gs_ref, b_ref, gw_ref, gb_ref, gms_ref,
             y_ref, st_ref, *, relu, last):
    gs = gs_ref[...]
    nf = float(N_NODES)
    mean = gs[0:1, :] / nf
    e2 = gs[1:2, :] / nf
    c = mean * gms_ref[...]
    var = e2 - 2.0 * mean * c + c * c
    y = gw_ref[...] * (xp_ref[...] - c) / jnp.sqrt(var + 1e-5) + gb_ref[...]
    if relu:
        y = jnp.maximum(y, 0.0)
    y_ref[...] = y

    @pl.when(pl.program_id(0) == 0)
    def _():
        st_ref[...] = jnp.zeros_like(st_ref)

    oh = _onehot(b_ref[...])
    if last:
        st_ref[...] += _segdot(oh, y)
    else:
        feats = y[:, POS_DIM:]
        v1 = jnp.sum(feats, axis=1, keepdims=True)
        v2 = jnp.sum(feats * feats, axis=1, keepdims=True)
        ones = jnp.ones_like(v1)
        vals = jnp.concatenate(
            [v1, v2, ones, jnp.zeros((v1.shape[0], 5), jnp.float32)], axis=1)
        st_ref[...] += _segdot(oh, vals)


def _ga_call(xp, gstats, b2, g, relu, last):
    full = lambda shp: pl.BlockSpec(shp, lambda i: (0, 0))
    return pl.pallas_call(
        functools.partial(_ga_body, relu=relu, last=last),
        grid=(GN,),
        in_specs=[
            pl.BlockSpec((BN, 8), lambda i: (i, 0)),
            full((8, 8)),
            pl.BlockSpec((BN, 1), lambda i: (i, 0)),
            full((1, 8)), full((1, 8)), full((1, 8)),
        ],
        out_specs=[
            pl.BlockSpec((BN, 8), lambda i: (i, 0)),
            pl.BlockSpec((NUM_GRAPHS, 8), lambda i: (0, 0)),
        ],
        out_shape=[
            jax.ShapeDtypeStruct((N_NODES, 8), jnp.float32),
            jax.ShapeDtypeStruct((NUM_GRAPHS, 8), jnp.float32),
        ],
    )(xp, gstats, b2,
      g["weight"][None, :], g["bias"][None, :], g["mean_scale"][None, :])


def _head_body(pool_ref, cnt_ref, w1t_ref, b1_ref, w2t_ref, b2_ref, out_ref):
    h = pool_ref[...] / jnp.maximum(cnt_ref[...], 1.0)
    h1 = jnp.maximum(
        jnp.dot(h, w1t_ref[...], preferred_element_type=jnp.float32, precision=lax.Precision.HIGHEST)
        + b1_ref[...], 0.0)
    out_ref[...] = (jnp.dot(h1, w2t_ref[...],
                            preferred_element_type=jnp.float32, precision=lax.Precision.HIGHEST) + b2_ref[...])


def _head_call(pool, cnt, fc):
    (w1, b1), (w2, b2) = fc
    full = lambda shp: pl.BlockSpec(shp, lambda: (0, 0))
    return pl.pallas_call(
        _head_body,
        in_specs=[full((NUM_GRAPHS, 8)), full((NUM_GRAPHS, 1)),
                  full((8, 32)), full((1, 32)),
                  full((32, 10)), full((1, 10))],
        out_specs=full((NUM_GRAPHS, 10)),
        out_shape=jax.ShapeDtypeStruct((NUM_GRAPHS, 10), jnp.float32),
    )(pool, cnt, w1.T, b1[None, :], w2.T, b2[None, :])


def kernel(x, edge_index, batch, edge_attr, params):
    src = edge_index[0]
    dst = edge_index[1]
    b2 = batch[:, None]
    src2d = src.reshape(SC_ROWS, 128)
    dst2d = dst.reshape(SC_ROWS, 128)
    z24 = jnp.zeros((N_PAD, 24), jnp.float32)

    lnstats = _ns_call(x, b2)
    cnt = lnstats[:, 2:3]

    x_cur = x
    for i in range(3):
        p = params["layers"][i]
        xs, xd = _sc_gather(x_cur, src2d, dst2d)
        msgc = _edge_call(xs, xd, edge_attr, p)
        acc = _sc_scatter(msgc, dst2d, z24)
        xp, gstats = _nu_call(x_cur, acc, b2, lnstats, p)
        last = i == 2
        x_cur, aux = _ga_call(xp, gstats, b2, params["gn"][i],
                              relu=not last, last=last)
        if not last:
            lnstats = aux
    return _head_call(aux, cnt, params["fc"])


# speedup vs baseline: 4.0063x; 2.4119x over previous
import functools

import jax
import jax.numpy as jnp
from jax import lax
from jax.experimental import pallas as pl
from jax.experimental.pallas import tpu as pltpu
from jax.experimental.pallas import tpu_sc as plsc

N_NODES = 50000
N_EDGES = 1600000
NUM_GRAPHS = 128
FEATS_DIM = 5
POS_DIM = 3
M_DIM = 16

BE = 2000
BN = 2000
GE = N_EDGES // BE
GN = N_NODES // BN


def _silu(v):
    return v * jax.nn.sigmoid(v)


def _dotT(w, x):
    return jnp.dot(w, x, preferred_element_type=jnp.float32,
                   precision=lax.Precision.HIGHEST)


def _edge_body(xs_ref, xd_ref, ea_ref, w1_ref, b1_ref, w2_ref, b2_ref,
               sw_ref, sb_ref, cw1_ref, cb1_ref, cw2_ref, cb2_ref,
               cs_ref, msg_ref):
    xsT = jnp.transpose(xs_ref[...])
    xdT = jnp.transpose(xd_ref[...])
    eaT = jnp.transpose(ea_ref[...])
    rel = xsT[0:POS_DIM, :] - xdT[0:POS_DIM, :]
    rd = jnp.sum(rel * rel, axis=0, keepdims=True)
    m_in = jnp.concatenate([xdT[POS_DIM:, :], xsT[POS_DIM:, :], eaT, rd],
                           axis=0)
    h = _silu(_dotT(w1_ref[...], m_in) + b1_ref[...])
    mij = _silu(_dotT(w2_ref[...], h) + b2_ref[...])
    ch = _silu(_dotT(cw1_ref[...], mij) + cb1_ref[...])
    cwij = _dotT(cw2_ref[...], ch) + cb2_ref[...]
    nrm = jnp.sqrt(jnp.maximum(rd, 1e-16))
    reln = rel / jnp.maximum(nrm, 1e-8) * cs_ref[0, 0]
    wv = cwij * reln
    gate = jax.nn.sigmoid(_dotT(sw_ref[...], mij) + sb_ref[...])
    msgT = jnp.concatenate(
        [mij * gate, wv, jnp.zeros((5, wv.shape[1]), jnp.float32)], axis=0)
    msg_ref[...] = jnp.transpose(msgT)


def _edge_call(xs, xd, ea, p):
    full = lambda shp: pl.BlockSpec(shp, lambda i: (0, 0))
    return pl.pallas_call(
        _edge_body,
        grid=(GE,),
        in_specs=[
            pl.BlockSpec((BE, 8), lambda i: (i, 0)),
            pl.BlockSpec((BE, 8), lambda i: (i, 0)),
            pl.BlockSpec((BE, 4), lambda i: (i, 0)),
            full((30, 15)), full((30, 1)),
            full((16, 30)), full((16, 1)),
            full((1, 16)), full((1, 1)),
            full((64, 16)), full((64, 1)),
            full((1, 64)), full((1, 1)),
            full((1, 1)),
        ],
        out_specs=pl.BlockSpec((BE, 24), lambda i: (i, 0)),
        out_shape=jax.ShapeDtypeStruct((N_EDGES, 24), jnp.float32),
    )(xs, xd, ea,
      p["edge_w1"], p["edge_b1"][:, None],
      p["edge_w2"], p["edge_b2"][:, None],
      p["soft_w"], p["soft_b"][:, None],
      p["coors_w1"], p["coors_b1"][:, None],
      p["coors_w2"], p["coors_b2"][:, None],
      p["coors_scale"].reshape(1, 1))


SC_ROWS = N_EDGES // 128
SC_K = 8
SC_FULL = SC_ROWS // SC_K
SC_T = SC_FULL // 32
SC_REM = SC_FULL - SC_T * 32
SC_TAIL = SC_ROWS - SC_FULL * SC_K
N_PAD = 50048
SC_NSLICE = N_PAD // 16


def _sc_scatter(msgc, dst2d, z24):
    mesh = plsc.VectorSubcoreMesh(core_axis_name="c", subcore_axis_name="s")

    @functools.partial(
        pl.kernel,
        out_type=jax.ShapeDtypeStruct((2, N_PAD, 24), jnp.float32),
        mesh=mesh,
        scratch_types=[
            pltpu.VMEM((SC_K, 128), jnp.int32),
            pltpu.VMEM((SC_K * 128, 24), jnp.float32),
            pltpu.VMEM_SHARED((N_PAD, 24), jnp.float32),
        ],
        compiler_params=pltpu.CompilerParams(use_tc_tiling_on_sc=False),
    )
    def k(msg_hbm, dst_hbm, z_hbm, out_hbm, idx_v, data_v, acc_sh):
        c = lax.axis_index("c")
        s = lax.axis_index("s")
        w = c * 16 + s
        r0 = s * SC_NSLICE
        pltpu.sync_copy(z_hbm.at[pl.ds(r0, SC_NSLICE)],
                        acc_sh.at[pl.ds(r0, SC_NSLICE)])
        plsc.subcore_barrier()

        def chunk(q, nrows):
            row = q * SC_K
            e0 = row * 128
            pltpu.sync_copy(dst_hbm.at[pl.ds(row, nrows)],
                            idx_v.at[pl.ds(0, nrows)])
            pltpu.sync_copy(msg_hbm.at[pl.ds(e0, nrows * 128)],
                            data_v.at[pl.ds(0, nrows * 128)])
            for j in range(nrows):
                pltpu.sync_copy(data_v.at[pl.ds(j * 128, 128)],
                                acc_sh.at[idx_v.at[j]], add=True)

        def body(t, carry):
            chunk(w + 32 * t, SC_K)
            return carry

        lax.fori_loop(0, SC_T, body, 0)

        @pl.when(w < SC_REM)
        def _():
            chunk(SC_T * 32 + w, SC_K)

        @pl.when(w == 31)
        def _():
            chunk(SC_FULL, SC_TAIL)

        plsc.subcore_barrier()
        pltpu.sync_copy(acc_sh.at[pl.ds(r0, SC_NSLICE)],
                        out_hbm.at[c, pl.ds(r0, SC_NSLICE)])

    return k(msgc, dst2d, z24)


def _sc_gather(x, src2d, dst2d):
    mesh = plsc.VectorSubcoreMesh(core_axis_name="c", subcore_axis_name="s")

    @functools.partial(
        pl.kernel,
        out_type=[jax.ShapeDtypeStruct((N_EDGES, 8), jnp.float32),
                  jax.ShapeDtypeStruct((N_EDGES, 8), jnp.float32)],
        mesh=mesh,
        scratch_types=[
            pltpu.VMEM((SC_K, 128), jnp.int32),
            pltpu.VMEM((SC_K, 128), jnp.int32),
            pltpu.VMEM((SC_K * 128, 8), jnp.float32),
            pltpu.VMEM((SC_K * 128, 8), jnp.float32),
            pltpu.SemaphoreType.DMA,
        ],
        compiler_params=pltpu.CompilerParams(use_tc_tiling_on_sc=False),
    )
    def k(x_hbm, src_hbm, dst_hbm, gs_hbm, gd_hbm,
          si_v, di_v, sb_v, db_v, sem):
        c = lax.axis_index("c")
        s = lax.axis_index("s")
        w = c * 16 + s

        def chunk(q, nrows):
            row = q * SC_K
            e0 = row * 128
            pltpu.sync_copy(src_hbm.at[pl.ds(row, nrows)],
                            si_v.at[pl.ds(0, nrows)])
            pltpu.sync_copy(dst_hbm.at[pl.ds(row, nrows)],
                            di_v.at[pl.ds(0, nrows)])
            hs = []
            for j in range(nrows):
                hs.append(pltpu.async_copy(
                    x_hbm.at[si_v.at[j]], sb_v.at[pl.ds(j * 128, 128)], sem))
                hs.append(pltpu.async_copy(
                    x_hbm.at[di_v.at[j]], db_v.at[pl.ds(j * 128, 128)], sem))
            for h in hs:
                h.wait()
            pltpu.sync_copy(sb_v.at[pl.ds(0, nrows * 128)],
                            gs_hbm.at[pl.ds(e0, nrows * 128)])
            pltpu.sync_copy(db_v.at[pl.ds(0, nrows * 128)],
                            gd_hbm.at[pl.ds(e0, nrows * 128)])

        def body(t, carry):
            chunk(w + 32 * t, SC_K)
            return carry

        lax.fori_loop(0, SC_T, body, 0)

        @pl.when(w < SC_REM)
        def _():
            chunk(SC_T * 32 + w, SC_K)

        @pl.when(w == 31)
        def _():
            chunk(SC_FULL, SC_TAIL)

    return k(x, src2d, dst2d)


def _onehot(b):
    g = lax.broadcasted_iota(jnp.int32, (b.shape[0], NUM_GRAPHS), 1)
    return (b == g).astype(jnp.float32)


def _segdot(oh, vals):
    return lax.dot_general(oh, vals, (((0,), (0,)), ((), ())),
                           preferred_element_type=jnp.float32, precision=lax.Precision.HIGHEST)


def _ns_body(x_ref, b_ref, out_ref):
    @pl.when(pl.program_id(0) == 0)
    def _():
        out_ref[...] = jnp.zeros_like(out_ref)

    feats = x_ref[...][:, POS_DIM:]
    v1 = jnp.sum(feats, axis=1, keepdims=True)
    v2 = jnp.sum(feats * feats, axis=1, keepdims=True)
    ones = jnp.ones_like(v1)
    vals = jnp.concatenate(
        [v1, v2, ones, jnp.zeros((v1.shape[0], 5), jnp.float32)], axis=1)
    out_ref[...] += _segdot(_onehot(b_ref[...]), vals)


def _ns_call(x, b2):
    return pl.pallas_call(
        _ns_body,
        grid=(GN,),
        in_specs=[
            pl.BlockSpec((BN, 8), lambda i: (i, 0)),
            pl.BlockSpec((BN, 1), lambda i: (i, 0)),
        ],
        out_specs=pl.BlockSpec((NUM_GRAPHS, 8), lambda i: (0, 0)),
        out_shape=jax.ShapeDtypeStruct((NUM_GRAPHS, 8), jnp.float32),
    )(x, b2)


def _nu_body(x_ref, acc_ref, b_ref, ls_ref, lnw_ref, lnb_ref,
             nw1t_ref, nb1_ref, nw2t_ref, nb2_ref, xp_ref, gs_ref):
    x = x_ref[...]
    acc = jnp.sum(acc_ref[...], axis=0)
    m_i = acc[:, 0:16]
    coors = x[:, 0:POS_DIM] + acc[:, 16:16 + POS_DIM]
    feats = x[:, POS_DIM:]

    ls = ls_ref[...]
    normv = jnp.maximum(ls[:, 2:3], 1.0) * float(FEATS_DIM)
    m = ls[:, 0:1] / normv
    var = ls[:, 1:2] / normv - m * m
    inv = lax.rsqrt(var + 1e-5)
    pg = jnp.concatenate(
        [m, inv, jnp.zeros((NUM_GRAPHS, 6), jnp.float32)], axis=1)
    pn = jnp.dot(_onehot(b_ref[...]), pg, preferred_element_type=jnp.float32, precision=lax.Precision.HIGHEST)
    feats_n = (feats - pn[:, 0:1]) * pn[:, 1:2] * lnw_ref[...] + lnb_ref[...]

    h2in = jnp.concatenate([feats_n, m_i], axis=1)
    h2 = _silu(jnp.dot(h2in, nw1t_ref[...], preferred_element_type=jnp.float32, precision=lax.Precision.HIGHEST)
               + nb1_ref[...])
    fo = feats + jnp.dot(h2, nw2t_ref[...],
                         preferred_element_type=jnp.float32, precision=lax.Precision.HIGHEST) + nb2_ref[...]
    xp = jnp.concatenate([coors, fo], axis=1)
    xp_ref[...] = xp

    @pl.when(pl.program_id(0) == 0)
    def _():
        gs_ref[...] = jnp.zeros_like(gs_ref)

    s1 = jnp.sum(xp, axis=0, keepdims=True)
    s2 = jnp.sum(xp * xp, axis=0, keepdims=True)
    gs_ref[...] += jnp.concatenate(
        [s1, s2, jnp.zeros((6, 8), jnp.float32)], axis=0)


def _nu_call(x, acc, b2, lnstats, p):
    full = lambda shp: pl.BlockSpec(shp, lambda i: tuple(0 for _ in shp))
    a = acc.shape[0]
    return pl.pallas_call(
        _nu_body,
        grid=(GN,),
        in_specs=[
            pl.BlockSpec((BN, 8), lambda i: (i, 0)),
            pl.BlockSpec((a, BN, 24), lambda i: (0, i, 0)),
            pl.BlockSpec((BN, 1), lambda i: (i, 0)),
            full((NUM_GRAPHS, 8)),
            full((1, 5)), full((1, 5)),
            full((21, 10)), full((1, 10)),
            full((10, 5)), full((1, 5)),
        ],
        out_specs=[
            pl.BlockSpec((BN, 8), lambda i: (i, 0)),
            pl.BlockSpec((8, 8), lambda i: (0, 0)),
        ],
        out_shape=[
            jax.ShapeDtypeStruct((N_NODES, 8), jnp.float32),
            jax.ShapeDtypeStruct((8, 8), jnp.float32),
        ],
    )(x, acc, b2, lnstats,
      p["ln_w"][None, :], p["ln_b"][None, :],
      p["node_w1"].T, p["node_b1"][None, :],
      p["node_w2"].T, p["node_b2"][None, :])


def _ga_body(xp_ref, gs_ref, b_ref, gw_ref, gb_ref, gms_ref,
             y_ref, st_ref, *, relu, last):
    gs = gs_ref[...]
    nf = float(N_NODES)
    mean = gs[0:1, :] / nf
    e2 = gs[1:2, :] / nf
    c = mean * gms_ref[...]
    var = e2 - 2.0 * mean * c + c * c
    y = gw_ref[...] * (xp_ref[...] - c) / jnp.sqrt(var + 1e-5) + gb_ref[...]
    if relu:
        y = jnp.maximum(y, 0.0)
    y_ref[...] = y

    @pl.when(pl.program_id(0) == 0)
    def _():
        st_ref[...] = jnp.zeros_like(st_ref)

    oh = _onehot(b_ref[...])
    if last:
        st_ref[...] += _segdot(oh, y)
    else:
        feats = y[:, POS_DIM:]
        v1 = jnp.sum(feats, axis=1, keepdims=True)
        v2 = jnp.sum(feats * feats, axis=1, keepdims=True)
        ones = jnp.ones_like(v1)
        vals = jnp.concatenate(
            [v1, v2, ones, jnp.zeros((v1.shape[0], 5), jnp.float32)], axis=1)
        st_ref[...] += _segdot(oh, vals)


def _ga_call(xp, gstats, b2, g, relu, last):
    full = lambda shp: pl.BlockSpec(shp, lambda i: (0, 0))
    return pl.pallas_call(
        functools.partial(_ga_body, relu=relu, last=last),
        grid=(GN,),
        in_specs=[
            pl.BlockSpec((BN, 8), lambda i: (i, 0)),
            full((8, 8)),
            pl.BlockSpec((BN, 1), lambda i: (i, 0)),
            full((1, 8)), full((1, 8)), full((1, 8)),
        ],
        out_specs=[
            pl.BlockSpec((BN, 8), lambda i: (i, 0)),
            pl.BlockSpec((NUM_GRAPHS, 8), lambda i: (0, 0)),
        ],
        out_shape=[
            jax.ShapeDtypeStruct((N_NODES, 8), jnp.float32),
            jax.ShapeDtypeStruct((NUM_GRAPHS, 8), jnp.float32),
        ],
    )(xp, gstats, b2,
      g["weight"][None, :], g["bias"][None, :], g["mean_scale"][None, :])


def _head_body(pool_ref, cnt_ref, w1t_ref, b1_ref, w2t_ref, b2_ref, out_ref):
    h = pool_ref[...] / jnp.maximum(cnt_ref[...], 1.0)
    h1 = jnp.maximum(
        jnp.dot(h, w1t_ref[...], preferred_element_type=jnp.float32, precision=lax.Precision.HIGHEST)
        + b1_ref[...], 0.0)
    out_ref[...] = (jnp.dot(h1, w2t_ref[...],
                            preferred_element_type=jnp.float32, precision=lax.Precision.HIGHEST) + b2_ref[...])


def _head_call(pool, cnt, fc):
    (w1, b1), (w2, b2) = fc
    full = lambda shp: pl.BlockSpec(shp, lambda: (0, 0))
    return pl.pallas_call(
        _head_body,
        in_specs=[full((NUM_GRAPHS, 8)), full((NUM_GRAPHS, 1)),
                  full((8, 32)), full((1, 32)),
                  full((32, 10)), full((1, 10))],
        out_specs=full((NUM_GRAPHS, 10)),
        out_shape=jax.ShapeDtypeStruct((NUM_GRAPHS, 10), jnp.float32),
    )(pool, cnt, w1.T, b1[None, :], w2.T, b2[None, :])


def kernel(x, edge_index, batch, edge_attr, params):
    src = edge_index[0]
    dst = edge_index[1]
    b2 = batch[:, None]
    src2d = src.reshape(SC_ROWS, 128)
    dst2d = dst.reshape(SC_ROWS, 128)
    z24 = jnp.zeros((N_PAD, 24), jnp.float32)

    lnstats = _ns_call(x, b2)
    cnt = lnstats[:, 2:3]

    x_cur = x
    for i in range(3):
        p = params["layers"][i]
        xs, xd = _sc_gather(x_cur, src2d, dst2d)
        msgc = _edge_call(xs, xd, edge_attr, p)
        acc = _sc_scatter(msgc, dst2d, z24)
        xp, gstats = _nu_call(x_cur, acc, b2, lnstats, p)
        last = i == 2
        x_cur, aux = _ga_call(xp, gstats, b2, params["gn"][i],
                              relu=not last, last=last)
        if not last:
            lnstats = aux
    return _head_call(aux, cnt, params["fc"])


# speedup vs baseline: 4.3363x; 1.0824x over previous
import functools

import jax
import jax.numpy as jnp
from jax import lax
from jax.experimental import pallas as pl
from jax.experimental.pallas import tpu as pltpu
from jax.experimental.pallas import tpu_sc as plsc

N_NODES = 50000
N_EDGES = 1600000
NUM_GRAPHS = 128
FEATS_DIM = 5
POS_DIM = 3
M_DIM = 16

BE = 8000
BN = 2000
GE = N_EDGES // BE
GN = N_NODES // BN


def _silu(v):
    return v * jax.nn.sigmoid(v)


def _dotT(w, x):
    return jnp.dot(w, x, preferred_element_type=jnp.float32,
                   precision=lax.Precision.HIGHEST)


def _edge_body(xs_ref, xd_ref, ea_ref, w1_ref, b1_ref, w2_ref, b2_ref,
               sw_ref, sb_ref, cw1_ref, cb1_ref, cw2_ref, cb2_ref,
               cs_ref, msg_ref):
    xsT = jnp.transpose(xs_ref[...])
    xdT = jnp.transpose(xd_ref[...])
    eaT = jnp.transpose(ea_ref[...])
    rel = xsT[0:POS_DIM, :] - xdT[0:POS_DIM, :]
    rd = jnp.sum(rel * rel, axis=0, keepdims=True)
    m_in = jnp.concatenate([xdT[POS_DIM:, :], xsT[POS_DIM:, :], eaT, rd],
                           axis=0)
    h = _silu(_dotT(w1_ref[...], m_in) + b1_ref[...])
    mij = _silu(_dotT(w2_ref[...], h) + b2_ref[...])
    ch = _silu(_dotT(cw1_ref[...], mij) + cb1_ref[...])
    cwij = _dotT(cw2_ref[...], ch) + cb2_ref[...]
    nrm = jnp.sqrt(jnp.maximum(rd, 1e-16))
    reln = rel / jnp.maximum(nrm, 1e-8) * cs_ref[0, 0]
    wv = cwij * reln
    gate = jax.nn.sigmoid(_dotT(sw_ref[...], mij) + sb_ref[...])
    msgT = jnp.concatenate(
        [mij * gate, wv, jnp.zeros((5, wv.shape[1]), jnp.float32)], axis=0)
    msg_ref[...] = jnp.transpose(msgT)


def _edge_call(xs, xd, ea, p):
    full = lambda shp: pl.BlockSpec(shp, lambda i: (0, 0))
    return pl.pallas_call(
        _edge_body,
        grid=(GE,),
        in_specs=[
            pl.BlockSpec((BE, 8), lambda i: (i, 0)),
            pl.BlockSpec((BE, 8), lambda i: (i, 0)),
            pl.BlockSpec((BE, 4), lambda i: (i, 0)),
            full((30, 15)), full((30, 1)),
            full((16, 30)), full((16, 1)),
            full((1, 16)), full((1, 1)),
            full((64, 16)), full((64, 1)),
            full((1, 64)), full((1, 1)),
            full((1, 1)),
        ],
        out_specs=pl.BlockSpec((BE, 24), lambda i: (i, 0)),
        out_shape=jax.ShapeDtypeStruct((N_EDGES, 24), jnp.float32),
    )(xs, xd, ea,
      p["edge_w1"], p["edge_b1"][:, None],
      p["edge_w2"], p["edge_b2"][:, None],
      p["soft_w"], p["soft_b"][:, None],
      p["coors_w1"], p["coors_b1"][:, None],
      p["coors_w2"], p["coors_b2"][:, None],
      p["coors_scale"].reshape(1, 1))


SC_ROWS = N_EDGES // 128
SC_K = 8
SC_FULL = SC_ROWS // SC_K
SC_T = SC_FULL // 32
SC_REM = SC_FULL - SC_T * 32
SC_TAIL = SC_ROWS - SC_FULL * SC_K
N_PAD = 50048
SC_NSLICE = N_PAD // 16


def _sc_scatter(msgc, dst2d, z24):
    mesh = plsc.VectorSubcoreMesh(core_axis_name="c", subcore_axis_name="s")

    @functools.partial(
        pl.kernel,
        out_type=jax.ShapeDtypeStruct((2, N_PAD, 24), jnp.float32),
        mesh=mesh,
        scratch_types=[
            pltpu.VMEM((SC_K, 128), jnp.int32),
            pltpu.VMEM((SC_K * 128, 24), jnp.float32),
            pltpu.VMEM_SHARED((N_PAD, 24), jnp.float32),
        ],
        compiler_params=pltpu.CompilerParams(use_tc_tiling_on_sc=False),
    )
    def k(msg_hbm, dst_hbm, z_hbm, out_hbm, idx_v, data_v, acc_sh):
        c = lax.axis_index("c")
        s = lax.axis_index("s")
        w = c * 16 + s
        r0 = s * SC_NSLICE
        pltpu.sync_copy(z_hbm.at[pl.ds(r0, SC_NSLICE)],
                        acc_sh.at[pl.ds(r0, SC_NSLICE)])
        plsc.subcore_barrier()

        def chunk(q, nrows):
            row = q * SC_K
            e0 = row * 128
            pltpu.sync_copy(dst_hbm.at[pl.ds(row, nrows)],
                            idx_v.at[pl.ds(0, nrows)])
            pltpu.sync_copy(msg_hbm.at[pl.ds(e0, nrows * 128)],
                            data_v.at[pl.ds(0, nrows * 128)])
            for j in range(nrows):
                pltpu.sync_copy(data_v.at[pl.ds(j * 128, 128)],
                                acc_sh.at[idx_v.at[j]], add=True)

        def body(t, carry):
            chunk(w + 32 * t, SC_K)
            return carry

        lax.fori_loop(0, SC_T, body, 0)

        @pl.when(w < SC_REM)
        def _():
            chunk(SC_T * 32 + w, SC_K)

        @pl.when(w == 31)
        def _():
            chunk(SC_FULL, SC_TAIL)

        plsc.subcore_barrier()
        pltpu.sync_copy(acc_sh.at[pl.ds(r0, SC_NSLICE)],
                        out_hbm.at[c, pl.ds(r0, SC_NSLICE)])

    return k(msgc, dst2d, z24)


def _sc_gather(x, src2d, dst2d):
    mesh = plsc.VectorSubcoreMesh(core_axis_name="c", subcore_axis_name="s")

    @functools.partial(
        pl.kernel,
        out_type=[jax.ShapeDtypeStruct((N_EDGES, 8), jnp.float32),
                  jax.ShapeDtypeStruct((N_EDGES, 8), jnp.float32)],
        mesh=mesh,
        scratch_types=[
            pltpu.VMEM((SC_K, 128), jnp.int32),
            pltpu.VMEM((SC_K, 128), jnp.int32),
            pltpu.VMEM((SC_K * 128, 8), jnp.float32),
            pltpu.VMEM((SC_K * 128, 8), jnp.float32),
            pltpu.SemaphoreType.DMA,
        ],
        compiler_params=pltpu.CompilerParams(use_tc_tiling_on_sc=False),
    )
    def k(x_hbm, src_hbm, dst_hbm, gs_hbm, gd_hbm,
          si_v, di_v, sb_v, db_v, sem):
        c = lax.axis_index("c")
        s = lax.axis_index("s")
        w = c * 16 + s

        def chunk(q, nrows):
            row = q * SC_K
            e0 = row * 128
            pltpu.sync_copy(src_hbm.at[pl.ds(row, nrows)],
                            si_v.at[pl.ds(0, nrows)])
            pltpu.sync_copy(dst_hbm.at[pl.ds(row, nrows)],
                            di_v.at[pl.ds(0, nrows)])
            hs = []
            for j in range(nrows):
                hs.append(pltpu.async_copy(
                    x_hbm.at[si_v.at[j]], sb_v.at[pl.ds(j * 128, 128)], sem))
                hs.append(pltpu.async_copy(
                    x_hbm.at[di_v.at[j]], db_v.at[pl.ds(j * 128, 128)], sem))
            for h in hs:
                h.wait()
            pltpu.sync_copy(sb_v.at[pl.ds(0, nrows * 128)],
                            gs_hbm.at[pl.ds(e0, nrows * 128)])
            pltpu.sync_copy(db_v.at[pl.ds(0, nrows * 128)],
                            gd_hbm.at[pl.ds(e0, nrows * 128)])

        def body(t, carry):
            chunk(w + 32 * t, SC_K)
            return carry

        lax.fori_loop(0, SC_T, body, 0)

        @pl.when(w < SC_REM)
        def _():
            chunk(SC_T * 32 + w, SC_K)

        @pl.when(w == 31)
        def _():
            chunk(SC_FULL, SC_TAIL)

    return k(x, src2d, dst2d)


def _onehot(b):
    g = lax.broadcasted_iota(jnp.int32, (b.shape[0], NUM_GRAPHS), 1)
    return (b == g).astype(jnp.float32)


def _segdot(oh, vals):
    return lax.dot_general(oh, vals, (((0,), (0,)), ((), ())),
                           preferred_element_type=jnp.float32, precision=lax.Precision.HIGHEST)


def _ns_body(x_ref, b_ref, out_ref):
    @pl.when(pl.program_id(0) == 0)
    def _():
        out_ref[...] = jnp.zeros_like(out_ref)

    feats = x_ref[...][:, POS_DIM:]
    v1 = jnp.sum(feats, axis=1, keepdims=True)
    v2 = jnp.sum(feats * feats, axis=1, keepdims=True)
    ones = jnp.ones_like(v1)
    vals = jnp.concatenate(
        [v1, v2, ones, jnp.zeros((v1.shape[0], 5), jnp.float32)], axis=1)
    out_ref[...] += _segdot(_onehot(b_ref[...]), vals)


def _ns_call(x, b2):
    return pl.pallas_call(
        _ns_body,
        grid=(GN,),
        in_specs=[
            pl.BlockSpec((BN, 8), lambda i: (i, 0)),
            pl.BlockSpec((BN, 1), lambda i: (i, 0)),
        ],
        out_specs=pl.BlockSpec((NUM_GRAPHS, 8), lambda i: (0, 0)),
        out_shape=jax.ShapeDtypeStruct((NUM_GRAPHS, 8), jnp.float32),
    )(x, b2)


def _nu_body(x_ref, acc_ref, b_ref, ls_ref, lnw_ref, lnb_ref,
             nw1t_ref, nb1_ref, nw2t_ref, nb2_ref, xp_ref, gs_ref):
    x = x_ref[...]
    acc = jnp.sum(acc_ref[...], axis=0)
    m_i = acc[:, 0:16]
    coors = x[:, 0:POS_DIM] + acc[:, 16:16 + POS_DIM]
    feats = x[:, POS_DIM:]

    ls = ls_ref[...]
    normv = jnp.maximum(ls[:, 2:3], 1.0) * float(FEATS_DIM)
    m = ls[:, 0:1] / normv
    var = ls[:, 1:2] / normv - m * m
    inv = lax.rsqrt(var + 1e-5)
    pg = jnp.concatenate(
        [m, inv, jnp.zeros((NUM_GRAPHS, 6), jnp.float32)], axis=1)
    pn = jnp.dot(_onehot(b_ref[...]), pg, preferred_element_type=jnp.float32, precision=lax.Precision.HIGHEST)
    feats_n = (feats - pn[:, 0:1]) * pn[:, 1:2] * lnw_ref[...] + lnb_ref[...]

    h2in = jnp.concatenate([feats_n, m_i], axis=1)
    h2 = _silu(jnp.dot(h2in, nw1t_ref[...], preferred_element_type=jnp.float32, precision=lax.Precision.HIGHEST)
               + nb1_ref[...])
    fo = feats + jnp.dot(h2, nw2t_ref[...],
                         preferred_element_type=jnp.float32, precision=lax.Precision.HIGHEST) + nb2_ref[...]
    xp = jnp.concatenate([coors, fo], axis=1)
    xp_ref[...] = xp

    @pl.when(pl.program_id(0) == 0)
    def _():
        gs_ref[...] = jnp.zeros_like(gs_ref)

    s1 = jnp.sum(xp, axis=0, keepdims=True)
    s2 = jnp.sum(xp * xp, axis=0, keepdims=True)
    gs_ref[...] += jnp.concatenate(
        [s1, s2, jnp.zeros((6, 8), jnp.float32)], axis=0)


def _nu_call(x, acc, b2, lnstats, p):
    full = lambda shp: pl.BlockSpec(shp, lambda i: tuple(0 for _ in shp))
    a = acc.shape[0]
    return pl.pallas_call(
        _nu_body,
        grid=(GN,),
        in_specs=[
            pl.BlockSpec((BN, 8), lambda i: (i, 0)),
            pl.BlockSpec((a, BN, 24), lambda i: (0, i, 0)),
            pl.BlockSpec((BN, 1), lambda i: (i, 0)),
            full((NUM_GRAPHS, 8)),
            full((1, 5)), full((1, 5)),
            full((21, 10)), full((1, 10)),
            full((10, 5)), full((1, 5)),
        ],
        out_specs=[
            pl.BlockSpec((BN, 8), lambda i: (i, 0)),
            pl.BlockSpec((8, 8), lambda i: (0, 0)),
        ],
        out_shape=[
            jax.ShapeDtypeStruct((N_NODES, 8), jnp.float32),
            jax.ShapeDtypeStruct((8, 8), jnp.float32),
        ],
    )(x, acc, b2, lnstats,
      p["ln_w"][None, :], p["ln_b"][None, :],
      p["node_w1"].T, p["node_b1"][None, :],
      p["node_w2"].T, p["node_b2"][None, :])


def _ga_body(xp_ref, gs_ref, b_ref, gw_ref, gb_ref, gms_ref,
             y_ref, st_ref, *, relu, last):
    gs = gs_ref[...]
    nf = float(N_NODES)
    mean = gs[0:1, :] / nf
    e2 = gs[1:2, :] / nf
    c = mean * gms_ref[...]
    var = e2 - 2.0 * mean * c + c * c
    y = gw_ref[...] * (xp_ref[...] - c) / jnp.sqrt(var + 1e-5) + gb_ref[...]
    if relu:
        y = jnp.maximum(y, 0.0)
    y_ref[...] = y

    @pl.when(pl.program_id(0) == 0)
    def _():
        st_ref[...] = jnp.zeros_like(st_ref)

    oh = _onehot(b_ref[...])
    if last:
        st_ref[...] += _segdot(oh, y)
    else:
        feats = y[:, POS_DIM:]
        v1 = jnp.sum(feats, axis=1, keepdims=True)
        v2 = jnp.sum(feats * feats, axis=1, keepdims=True)
        ones = jnp.ones_like(v1)
        vals = jnp.concatenate(
            [v1, v2, ones, jnp.zeros((v1.shape[0], 5), jnp.float32)], axis=1)
        st_ref[...] += _segdot(oh, vals)


def _ga_call(xp, gstats, b2, g, relu, last):
    full = lambda shp: pl.BlockSpec(shp, lambda i: (0, 0))
    return pl.pallas_call(
        functools.partial(_ga_body, relu=relu, last=last),
        grid=(GN,),
        in_specs=[
            pl.BlockSpec((BN, 8), lambda i: (i, 0)),
            full((8, 8)),
            pl.BlockSpec((BN, 1), lambda i: (i, 0)),
            full((1, 8)), full((1, 8)), full((1, 8)),
        ],
        out_specs=[
            pl.BlockSpec((BN, 8), lambda i: (i, 0)),
            pl.BlockSpec((NUM_GRAPHS, 8), lambda i: (0, 0)),
        ],
        out_shape=[
            jax.ShapeDtypeStruct((N_NODES, 8), jnp.float32),
            jax.ShapeDtypeStruct((NUM_GRAPHS, 8), jnp.float32),
        ],
    )(xp, gstats, b2,
      g["weight"][None, :], g["bias"][None, :], g["mean_scale"][None, :])


def _head_body(pool_ref, cnt_ref, w1t_ref, b1_ref, w2t_ref, b2_ref, out_ref):
    h = pool_ref[...] / jnp.maximum(cnt_ref[...], 1.0)
    h1 = jnp.maximum(
        jnp.dot(h, w1t_ref[...], preferred_element_type=jnp.float32, precision=lax.Precision.HIGHEST)
        + b1_ref[...], 0.0)
    out_ref[...] = (jnp.dot(h1, w2t_ref[...],
                            preferred_element_type=jnp.float32, precision=lax.Precision.HIGHEST) + b2_ref[...])


def _head_call(pool, cnt, fc):
    (w1, b1), (w2, b2) = fc
    full = lambda shp: pl.BlockSpec(shp, lambda: (0, 0))
    return pl.pallas_call(
        _head_body,
        in_specs=[full((NUM_GRAPHS, 8)), full((NUM_GRAPHS, 1)),
                  full((8, 32)), full((1, 32)),
                  full((32, 10)), full((1, 10))],
        out_specs=full((NUM_GRAPHS, 10)),
        out_shape=jax.ShapeDtypeStruct((NUM_GRAPHS, 10), jnp.float32),
    )(pool, cnt, w1.T, b1[None, :], w2.T, b2[None, :])


def kernel(x, edge_index, batch, edge_attr, params):
    src = edge_index[0]
    dst = edge_index[1]
    b2 = batch[:, None]
    src2d = src.reshape(SC_ROWS, 128)
    dst2d = dst.reshape(SC_ROWS, 128)
    z24 = jnp.zeros((N_PAD, 24), jnp.float32)

    lnstats = _ns_call(x, b2)
    cnt = lnstats[:, 2:3]

    x_cur = x
    for i in range(3):
        p = params["layers"][i]
        xs, xd = _sc_gather(x_cur, src2d, dst2d)
        msgc = _edge_call(xs, xd, edge_attr, p)
        acc = _sc_scatter(msgc, dst2d, z24)
        xp, gstats = _nu_call(x_cur, acc, b2, lnstats, p)
        last = i == 2
        x_cur, aux = _ga_call(xp, gstats, b2, params["gn"][i],
                              relu=not last, last=last)
        if not last:
            lnstats = aux
    return _head_call(aux, cnt, params["fc"])
